# Initial kernel scaffold; baseline (speedup 1.0000x reference)
import jax
import jax.numpy as jnp
from jax.experimental import pallas as pl


def kernel(prediction, target, mask):
    raise NotImplementedError("write your pallas kernel here")


# speedup vs baseline: 19.8069x; 19.8069x over previous
import functools

import jax
import jax.numpy as jnp
from jax.experimental import pallas as pl
from jax.experimental.pallas import tpu as pltpu

_SENTINEL = 0x7F800000


def _trimmed_mae_kernel(pred_ref, targ_ref, mask_ref, out_ref, ebits_ref):
    p = pred_ref[...]
    t = targ_ref[...]
    m = mask_ref[...] > 0
    err = jnp.abs(p - t)
    ebits = jnp.where(m, jax.lax.bitcast_convert_type(err, jnp.int32),
                      jnp.int32(_SENTINEL))
    ebits_ref[...] = ebits

    mf = m.astype(jnp.float32)
    M = jnp.sum(mf, axis=1, keepdims=True)
    k_f = jnp.floor(M * jnp.float32(0.8))
    k = k_f.astype(jnp.int32)

    def body(_, carry):
        lo, hi = carry
        mid = lo + (hi - lo) // 2
        x = ebits_ref[...]
        cnt = jnp.sum((x <= mid).astype(jnp.int32), axis=1, keepdims=True)
        ge = cnt >= k
        return jnp.where(ge, lo, mid + 1), jnp.where(ge, mid, hi)

    B = p.shape[0]
    lo0 = jnp.zeros((B, 1), jnp.int32)
    hi0 = jnp.full((B, 1), _SENTINEL, jnp.int32)
    _, kth = jax.lax.fori_loop(0, 31, body, (lo0, hi0))

    x = ebits_ref[...]
    below = x < kth
    s = jnp.sum(jnp.where(below, jax.lax.bitcast_convert_type(x, jnp.float32), 0.0),
                axis=1, keepdims=True)
    c = jnp.sum(below.astype(jnp.float32), axis=1, keepdims=True)
    t_val = jax.lax.bitcast_convert_type(kth, jnp.float32)
    image_loss = s + (k_f - c) * t_val

    divisor = jnp.sum(M * jnp.float32(0.8), axis=0, keepdims=True)
    total = jnp.sum(image_loss, axis=0, keepdims=True)
    loss = jnp.where(divisor == 0.0, jnp.float32(0.0),
                     total / jnp.maximum(divisor, jnp.float32(1e-12)))
    out_ref[...] = loss


@jax.jit
def kernel(prediction, target, mask):
    B = prediction.shape[0]
    HW = prediction.shape[1] * prediction.shape[2]
    pred = prediction.reshape(B, HW)
    targ = target.reshape(B, HW)
    mflat = mask.reshape(B, HW)
    out = pl.pallas_call(
        _trimmed_mae_kernel,
        out_shape=jax.ShapeDtypeStruct((1, 1), jnp.float32),
        scratch_shapes=[pltpu.VMEM((B, HW), jnp.int32)],
    )(pred, targ, mflat)
    return out.reshape(())
